# baseline (speedup 1.0000x reference)
import functools

import jax
import jax.numpy as jnp
from jax import lax
from jax.experimental import pallas as pl
from jax.experimental.pallas import tpu as pltpu
from jax.experimental.pallas import tpu_sc as plsc

VOCAB = 1000
BATCH = 4096
SEQ = 20
NC, NS = 2, 16
NW = NC * NS
NB_PER_W = BATCH // NW
NBUF = 2


def _sc_gather(table, idx3):
    mesh = plsc.VectorSubcoreMesh(core_axis_name="c", subcore_axis_name="s")

    @functools.partial(
        pl.kernel,
        mesh=mesh,
        compiler_params=pltpu.CompilerParams(use_tc_tiling_on_sc=False),
        out_type=jax.ShapeDtypeStruct((BATCH, SEQ, VOCAB), jnp.float32),
        scratch_types=[
            pltpu.VMEM_SHARED((VOCAB, VOCAB), jnp.float32),
            pltpu.VMEM((NB_PER_W, SEQ), jnp.int32),
            pltpu.VMEM((SEQ, VOCAB), jnp.float32),
            pltpu.VMEM((SEQ, VOCAB), jnp.float32),
            pltpu.SemaphoreType.DMA,
            pltpu.SemaphoreType.DMA,
            pltpu.SemaphoreType.DMA,
            pltpu.SemaphoreType.DMA,
        ],
    )
    def k(table_hbm, idx_hbm, out_hbm,
          table_sp, idx_v, buf0, buf1,
          g0, g1, o0, o1):
        bufs = (buf0, buf1)
        gsems = (g0, g1)
        osems = (o0, o1)
        sid = lax.axis_index("s")
        wid = sid * NC + lax.axis_index("c")
        base = wid * NB_PER_W
        pltpu.sync_copy(idx_hbm.at[wid], idx_v)

        tr = sid * 63

        @pl.when(sid < NS - 1)
        def _():
            pltpu.sync_copy(table_hbm.at[pl.ds(tr, 63)],
                            table_sp.at[pl.ds(tr, 63)])

        @pl.when(sid == NS - 1)
        def _():
            pltpu.sync_copy(table_hbm.at[pl.ds(tr, VOCAB - 63 * (NS - 1))],
                            table_sp.at[pl.ds(tr, VOCAB - 63 * (NS - 1))])

        plsc.subcore_barrier()

        def gather_desc(c, b):
            return pltpu.make_async_copy(
                table_sp.at[idx_v.at[c]], bufs[b], gsems[b])

        def ocopy_desc(c, b):
            return pltpu.make_async_copy(bufs[b], out_hbm.at[base + c],
                                         osems[b])

        for b in range(NBUF):
            gather_desc(b, b).start()

        def body(s, _):
            c0 = s * NBUF
            for b in range(NBUF):
                gather_desc(c0 + b, b).wait()
                ocopy_desc(c0 + b, b).start()
            for b in range(NBUF):
                ocopy_desc(c0 + b, b).wait()
                gather_desc(c0 + NBUF + b, b).start()
            return _

        lax.fori_loop(0, NB_PER_W // NBUF - 1, body, None)

        cl = NB_PER_W - NBUF
        for b in range(NBUF):
            gather_desc(cl + b, b).wait()
            ocopy_desc(cl + b, b).start()
        for b in range(NBUF):
            ocopy_desc(cl + b, b).wait()

    return k(table, idx3)


@jax.jit
def kernel(idx, logits_table):
    idx3 = idx.reshape(NW, NB_PER_W, SEQ).astype(jnp.int32)
    return _sc_gather(logits_table, idx3)


# speedup vs baseline: 1.0973x; 1.0973x over previous
import functools

import jax
import jax.numpy as jnp
from jax import lax
from jax.experimental import pallas as pl
from jax.experimental.pallas import tpu as pltpu
from jax.experimental.pallas import tpu_sc as plsc

VOCAB = 1000
B_TOTAL = 4096 * 20
NC, NS = 2, 16
NW = NC * NS
B_PER_W = B_TOTAL // NW
CHUNK = 16
NBUF = 4
NCHUNK = B_PER_W // CHUNK
NGROUP = NCHUNK // NBUF


def _sc_gather(table, idx_flat):
    mesh = plsc.VectorSubcoreMesh(core_axis_name="c", subcore_axis_name="s")

    @functools.partial(
        pl.kernel,
        mesh=mesh,
        compiler_params=pltpu.CompilerParams(use_tc_tiling_on_sc=False),
        out_type=jax.ShapeDtypeStruct((B_TOTAL, VOCAB), jnp.float32),
        scratch_types=[
            pltpu.VMEM_SHARED((VOCAB, VOCAB), jnp.float32),
            pltpu.VMEM((B_PER_W,), jnp.int32),
        ] + [pltpu.VMEM((CHUNK, VOCAB), jnp.float32)] * NBUF
          + [pltpu.SemaphoreType.DMA] * (2 * NBUF),
    )
    def k(table_hbm, idx_hbm, out_hbm, table_sp, idx_v, *rest):
        bufs = rest[:NBUF]
        gsems = rest[NBUF:2 * NBUF]
        osems = rest[2 * NBUF:]
        sid = lax.axis_index("s")
        wid = sid * NC + lax.axis_index("c")
        base = wid * B_PER_W
        pltpu.sync_copy(idx_hbm.at[pl.ds(base, B_PER_W)], idx_v)

        tr = sid * 63

        @pl.when(sid < NS - 1)
        def _():
            pltpu.sync_copy(table_hbm.at[pl.ds(tr, 63)],
                            table_sp.at[pl.ds(tr, 63)])

        @pl.when(sid == NS - 1)
        def _():
            pltpu.sync_copy(table_hbm.at[pl.ds(tr, VOCAB - 63 * (NS - 1))],
                            table_sp.at[pl.ds(tr, VOCAB - 63 * (NS - 1))])

        plsc.subcore_barrier()

        def gather_desc(c, b):
            idx_slice = idx_v.at[pl.ds(c * CHUNK, CHUNK)]
            return pltpu.make_async_copy(
                table_sp.at[idx_slice], bufs[b], gsems[b])

        def ocopy_desc(c, b):
            return pltpu.make_async_copy(
                bufs[b], out_hbm.at[pl.ds(base + c * CHUNK, CHUNK)], osems[b])

        for b in range(NBUF):
            gather_desc(b, b).start()

        def body(s, _):
            c0 = s * NBUF
            for b in range(NBUF):
                gather_desc(c0 + b, b).wait()
                ocopy_desc(c0 + b, b).start()
            for b in range(NBUF):
                ocopy_desc(c0 + b, b).wait()
                gather_desc(c0 + NBUF + b, b).start()
            return _

        lax.fori_loop(0, NGROUP - 1, body, None)

        cl = (NGROUP - 1) * NBUF
        for b in range(NBUF):
            gather_desc(cl + b, b).wait()
            ocopy_desc(cl + b, b).start()
        for b in range(NBUF):
            ocopy_desc(cl + b, b).wait()

    return k(table, idx_flat)


@jax.jit
def kernel(idx, logits_table):
    idx_flat = idx.reshape(-1).astype(jnp.int32)
    out = _sc_gather(logits_table, idx_flat)
    return out.reshape(idx.shape[0], idx.shape[1], VOCAB)


# speedup vs baseline: 1.0997x; 1.0022x over previous
import functools

import jax
import jax.numpy as jnp
from jax import lax
from jax.experimental import pallas as pl
from jax.experimental.pallas import tpu as pltpu
from jax.experimental.pallas import tpu_sc as plsc

VOCAB = 1000
B_TOTAL = 4096 * 20
NC, NS = 2, 16
NW = NC * NS
B_PER_W = B_TOTAL // NW
CHUNK = 8
NBUF = 8
NCHUNK = B_PER_W // CHUNK
NGROUP = NCHUNK // NBUF


def _sc_gather(table, idx_flat):
    mesh = plsc.VectorSubcoreMesh(core_axis_name="c", subcore_axis_name="s")

    @functools.partial(
        pl.kernel,
        mesh=mesh,
        compiler_params=pltpu.CompilerParams(use_tc_tiling_on_sc=False),
        out_type=jax.ShapeDtypeStruct((B_TOTAL, VOCAB), jnp.float32),
        scratch_types=[
            pltpu.VMEM_SHARED((VOCAB, VOCAB), jnp.float32),
            pltpu.VMEM((B_PER_W,), jnp.int32),
        ] + [pltpu.VMEM((CHUNK, VOCAB), jnp.float32)] * NBUF
          + [pltpu.SemaphoreType.DMA] * (2 * NBUF),
    )
    def k(table_hbm, idx_hbm, out_hbm, table_sp, idx_v, *rest):
        bufs = rest[:NBUF]
        gsems = rest[NBUF:2 * NBUF]
        osems = rest[2 * NBUF:]
        sid = lax.axis_index("s")
        wid = sid * NC + lax.axis_index("c")
        base = wid * B_PER_W
        pltpu.sync_copy(idx_hbm.at[pl.ds(base, B_PER_W)], idx_v)

        tr = sid * 63

        @pl.when(sid < NS - 1)
        def _():
            pltpu.sync_copy(table_hbm.at[pl.ds(tr, 63)],
                            table_sp.at[pl.ds(tr, 63)])

        @pl.when(sid == NS - 1)
        def _():
            pltpu.sync_copy(table_hbm.at[pl.ds(tr, VOCAB - 63 * (NS - 1))],
                            table_sp.at[pl.ds(tr, VOCAB - 63 * (NS - 1))])

        plsc.subcore_barrier()

        def gather_desc(c, b):
            idx_slice = idx_v.at[pl.ds(c * CHUNK, CHUNK)]
            return pltpu.make_async_copy(
                table_sp.at[idx_slice], bufs[b], gsems[b])

        def ocopy_desc(c, b):
            return pltpu.make_async_copy(
                bufs[b], out_hbm.at[pl.ds(base + c * CHUNK, CHUNK)], osems[b])

        for b in range(NBUF):
            gather_desc(b, b).start()

        def body(s, _):
            c0 = s * NBUF
            for b in range(NBUF):
                gather_desc(c0 + b, b).wait()
                ocopy_desc(c0 + b, b).start()
            for b in range(NBUF):
                ocopy_desc(c0 + b, b).wait()
                gather_desc(c0 + NBUF + b, b).start()
            return _

        lax.fori_loop(0, NGROUP - 1, body, None)

        cl = (NGROUP - 1) * NBUF
        for b in range(NBUF):
            gather_desc(cl + b, b).wait()
            ocopy_desc(cl + b, b).start()
        for b in range(NBUF):
            ocopy_desc(cl + b, b).wait()

    return k(table, idx_flat)


@jax.jit
def kernel(idx, logits_table):
    idx_flat = idx.reshape(-1).astype(jnp.int32)
    out = _sc_gather(logits_table, idx_flat)
    return out.reshape(idx.shape[0], idx.shape[1], VOCAB)
